# Initial kernel scaffold; baseline (speedup 1.0000x reference)
import jax
import jax.numpy as jnp
from jax.experimental import pallas as pl


def kernel(x, Wg, W1, b1, W2, b2):
    raise NotImplementedError("write your pallas kernel here")


# speedup vs baseline: 2.9129x; 2.9129x over previous
import jax
import jax.numpy as jnp
from jax.experimental import pallas as pl
from jax.experimental.pallas import tpu as pltpu

E = 8
TOP_K = 2
TILE = 256
FF_CHUNK = 2048


def _ffn_body(te_ref, tl_ref, x_ref, W1_ref, b1_ref, W2_ref, b2_ref, o_ref):
    i = pl.program_id(0)
    ff = pl.program_id(1)

    @pl.when(tl_ref[i] != 0)
    def _compute():
        h = jnp.dot(x_ref[...], W1_ref[0], preferred_element_type=jnp.float32)
        h = h + b1_ref[0, 0][None, :]
        h = 0.5 * h * (1.0 + jax.lax.erf(h * 0.7071067811865476))
        y = jnp.dot(h, W2_ref[0], preferred_element_type=jnp.float32)

        @pl.when(ff == 0)
        def _init():
            o_ref[...] = y + b2_ref[0, 0][None, :]

        @pl.when(ff != 0)
        def _acc():
            o_ref[...] += y


def _grouped_ffn(x_sorted, W1, b1, W2, b2, tile_expert, tile_live, n_pad):
    d_model = x_sorted.shape[1]
    d_ff = W1.shape[2]
    nt = n_pad // TILE
    nff = d_ff // FF_CHUNK
    grid_spec = pltpu.PrefetchScalarGridSpec(
        num_scalar_prefetch=2,
        grid=(nt, nff),
        in_specs=[
            pl.BlockSpec((TILE, d_model), lambda i, ff, te, tl: (i, 0)),
            pl.BlockSpec((1, d_model, FF_CHUNK),
                         lambda i, ff, te, tl: (te[i], 0, ff)),
            pl.BlockSpec((1, 1, FF_CHUNK), lambda i, ff, te, tl: (te[i], 0, ff)),
            pl.BlockSpec((1, FF_CHUNK, d_model),
                         lambda i, ff, te, tl: (te[i], ff, 0)),
            pl.BlockSpec((1, 1, d_model), lambda i, ff, te, tl: (te[i], 0, 0)),
        ],
        out_specs=pl.BlockSpec((TILE, d_model), lambda i, ff, te, tl: (i, 0)),
    )
    return pl.pallas_call(
        _ffn_body,
        grid_spec=grid_spec,
        out_shape=jax.ShapeDtypeStruct((n_pad, d_model), jnp.float32),
        compiler_params=pltpu.CompilerParams(
            dimension_semantics=("arbitrary", "arbitrary")),
    )(tile_expert, tile_live, x_sorted, W1,
      b1.reshape(b1.shape[0], 1, d_ff), W2,
      b2.reshape(b2.shape[0], 1, d_model))


def kernel(x, Wg, W1, b1, W2, b2):
    bsz, seq, d = x.shape
    x_flat = x.reshape(-1, d)
    n_tok = bsz * seq
    n_pairs = n_tok * TOP_K
    nt = n_pairs // TILE + E
    n_pad = nt * TILE

    gate_scores = x_flat @ Wg
    gate_probs = jax.nn.softmax(gate_scores, axis=-1)
    top_k_probs, top_k_idx = jax.lax.top_k(gate_probs, TOP_K)
    top_k_probs = top_k_probs / jnp.sum(top_k_probs, axis=-1, keepdims=True)

    e_flat = top_k_idx.reshape(-1).astype(jnp.int32)
    p_flat = top_k_probs.reshape(-1)
    onehot = (e_flat[:, None] == jnp.arange(E, dtype=jnp.int32)[None, :])
    onehot = onehot.astype(jnp.int32)
    rank = (jnp.cumsum(onehot, axis=0) - onehot)
    rank = (rank * onehot).sum(axis=1)
    counts = onehot.sum(axis=0)
    padded = ((counts + TILE - 1) // TILE) * TILE
    starts = jnp.concatenate(
        [jnp.zeros((1,), jnp.int32), jnp.cumsum(padded).astype(jnp.int32)])[:E]
    pos = starts[e_flat] + rank

    start_tile = starts // TILE
    tile_ids = jnp.arange(nt, dtype=jnp.int32)
    tile_expert = (tile_ids[:, None] >= start_tile[None, :]).sum(
        axis=1).astype(jnp.int32) - 1
    total_tiles = jnp.sum(padded) // TILE
    tile_live = (tile_ids < total_tiles).astype(jnp.int32)
    e_last = jnp.max(jnp.where(counts > 0, jnp.arange(E, dtype=jnp.int32), 0))
    tile_expert = jnp.where(tile_live != 0, tile_expert, e_last)

    pair_tok = (jnp.arange(n_pairs, dtype=jnp.int32) // TOP_K)
    row_ids = jnp.zeros((n_pad,), jnp.int32).at[pos].set(pair_tok)
    x_sorted = x_flat[row_ids]

    y_sorted = _grouped_ffn(x_sorted, W1, b1, W2, b2,
                            tile_expert, tile_live, n_pad)

    pos2 = pos.reshape(n_tok, TOP_K)
    out = (top_k_probs[:, 0:1] * y_sorted[pos2[:, 0]]
           + top_k_probs[:, 1:2] * y_sorted[pos2[:, 1]])
    return out.reshape(bsz, seq, d)


# speedup vs baseline: 2.9757x; 1.0216x over previous
import jax
import jax.numpy as jnp
from jax.experimental import pallas as pl
from jax.experimental.pallas import tpu as pltpu

E = 8
TOP_K = 2
TILE = 256
FF_CHUNK = 4096


def _ffn_body(te_ref, tl_ref, x_ref, W1_ref, b1_ref, W2_ref, b2_ref, o_ref):
    i = pl.program_id(0)
    ff = pl.program_id(1)

    @pl.when(tl_ref[i] != 0)
    def _compute():
        h = jnp.dot(x_ref[...], W1_ref[0], preferred_element_type=jnp.float32)
        h = h + b1_ref[0, 0][None, :]
        h = 0.5 * h * (1.0 + jax.lax.erf(h * 0.7071067811865476))
        y = jnp.dot(h.astype(jnp.bfloat16), W2_ref[0],
                    preferred_element_type=jnp.float32)

        @pl.when(ff == 0)
        def _init():
            o_ref[...] = y + b2_ref[0, 0][None, :]

        @pl.when(ff != 0)
        def _acc():
            o_ref[...] += y


def _grouped_ffn(x_sorted, W1, b1, W2, b2, tile_expert, tile_live, n_pad):
    d_model = x_sorted.shape[1]
    d_ff = W1.shape[2]
    nt = n_pad // TILE
    nff = d_ff // FF_CHUNK
    grid_spec = pltpu.PrefetchScalarGridSpec(
        num_scalar_prefetch=2,
        grid=(nt, nff),
        in_specs=[
            pl.BlockSpec((TILE, d_model), lambda i, ff, te, tl: (i, 0)),
            pl.BlockSpec((1, d_model, FF_CHUNK),
                         lambda i, ff, te, tl: (te[i], 0, ff)),
            pl.BlockSpec((1, 1, FF_CHUNK), lambda i, ff, te, tl: (te[i], 0, ff)),
            pl.BlockSpec((1, FF_CHUNK, d_model),
                         lambda i, ff, te, tl: (te[i], ff, 0)),
            pl.BlockSpec((1, 1, d_model), lambda i, ff, te, tl: (te[i], 0, 0)),
        ],
        out_specs=pl.BlockSpec((TILE, d_model), lambda i, ff, te, tl: (i, 0)),
    )
    return pl.pallas_call(
        _ffn_body,
        grid_spec=grid_spec,
        out_shape=jax.ShapeDtypeStruct((n_pad, d_model), jnp.float32),
        compiler_params=pltpu.CompilerParams(
            dimension_semantics=("arbitrary", "arbitrary")),
    )(tile_expert, tile_live, x_sorted, W1.astype(jnp.bfloat16),
      b1.reshape(b1.shape[0], 1, d_ff), W2.astype(jnp.bfloat16),
      b2.reshape(b2.shape[0], 1, d_model))


def kernel(x, Wg, W1, b1, W2, b2):
    bsz, seq, d = x.shape
    x_flat = x.reshape(-1, d)
    n_tok = bsz * seq
    n_pairs = n_tok * TOP_K
    nt = n_pairs // TILE + E
    n_pad = nt * TILE

    gate_scores = x_flat @ Wg
    gate_probs = jax.nn.softmax(gate_scores, axis=-1)
    top_k_probs, top_k_idx = jax.lax.top_k(gate_probs, TOP_K)
    top_k_probs = top_k_probs / jnp.sum(top_k_probs, axis=-1, keepdims=True)

    e_flat = top_k_idx.reshape(-1).astype(jnp.int32)
    p_flat = top_k_probs.reshape(-1)
    onehot = (e_flat[:, None] == jnp.arange(E, dtype=jnp.int32)[None, :])
    onehot = onehot.astype(jnp.int32)
    rank = (jnp.cumsum(onehot, axis=0) - onehot)
    rank = (rank * onehot).sum(axis=1)
    counts = onehot.sum(axis=0)
    padded = ((counts + TILE - 1) // TILE) * TILE
    starts = jnp.concatenate(
        [jnp.zeros((1,), jnp.int32), jnp.cumsum(padded).astype(jnp.int32)])[:E]
    pos = starts[e_flat] + rank

    start_tile = starts // TILE
    tile_ids = jnp.arange(nt, dtype=jnp.int32)
    tile_expert = (tile_ids[:, None] >= start_tile[None, :]).sum(
        axis=1).astype(jnp.int32) - 1
    total_tiles = jnp.sum(padded) // TILE
    tile_live = (tile_ids < total_tiles).astype(jnp.int32)
    e_last = jnp.max(jnp.where(counts > 0, jnp.arange(E, dtype=jnp.int32), 0))
    tile_expert = jnp.where(tile_live != 0, tile_expert, e_last)

    pair_tok = (jnp.arange(n_pairs, dtype=jnp.int32) // TOP_K)
    row_ids = jnp.zeros((n_pad,), jnp.int32).at[pos].set(pair_tok)
    x_sorted = x_flat.astype(jnp.bfloat16)[row_ids]

    y_sorted = _grouped_ffn(x_sorted, W1, b1, W2, b2,
                            tile_expert, tile_live, n_pad)

    pos2 = pos.reshape(n_tok, TOP_K)
    out = (top_k_probs[:, 0:1] * y_sorted[pos2[:, 0]]
           + top_k_probs[:, 1:2] * y_sorted[pos2[:, 1]])
    return out.reshape(bsz, seq, d)
